# Initial kernel scaffold; baseline (speedup 1.0000x reference)
import jax
import jax.numpy as jnp
from jax.experimental import pallas as pl


def kernel(x, edge_index, batch, W_gcn, b_gcn, W1, b1, W2, b2):
    raise NotImplementedError("write your pallas kernel here")


# speedup vs baseline: 62.1416x; 62.1416x over previous
import jax
import jax.numpy as jnp
from jax import lax
from jax.experimental import pallas as pl
from jax.experimental.pallas import tpu as pltpu
from jax.experimental.pallas import tpu_sc as plsc

N = 65536
E = 1048576
D = 32
H = 16
NC, NS = 2, 16
CH = 2048
NPT = N // NS

_mesh = plsc.VectorSubcoreMesh(
    core_axis_name="c", subcore_axis_name="s", num_cores=NC, num_subcores=NS
)


def _deg_body(col_hbm, deg_out, col_v, ones_v, zero_v, deg_sh):
    c = lax.axis_index("c")
    s = lax.axis_index("s")

    def fill(i, _):
        ones_v[pl.ds(i * 16, 16)] = jnp.full((16,), 1.0, jnp.float32)
        zero_v[pl.ds(i * 16, 16)] = jnp.zeros((16,), jnp.float32)
        return 0

    lax.fori_loop(0, CH // 16, fill, 0)

    def zfill(i, _):
        zero_v[pl.ds(CH + i * 16, 16)] = jnp.zeros((16,), jnp.float32)
        return 0

    lax.fori_loop(0, (NPT - CH) // 16, zfill, 0)
    pltpu.sync_copy(zero_v, deg_sh.at[pl.ds(s * NPT, NPT)])
    plsc.subcore_barrier()

    ept = E // NC // NS
    base0 = c * (E // NC) + s * ept

    def step(k, _):
        pltpu.sync_copy(col_hbm.at[pl.ds(base0 + k * CH, CH)], col_v)
        pltpu.sync_copy(ones_v, deg_sh.at[col_v], add=True)
        return 0

    lax.fori_loop(0, ept // CH, step, 0)
    plsc.subcore_barrier()
    pltpu.sync_copy(deg_sh.at[pl.ds(s * NPT, NPT)], deg_out.at[c, pl.ds(s * NPT, NPT)])


_deg = pl.kernel(
    _deg_body,
    out_type=jax.ShapeDtypeStruct((NC, N), jnp.float32),
    mesh=_mesh,
    compiler_params=pltpu.CompilerParams(use_tc_tiling_on_sc=False),
    scratch_types=[
        pltpu.VMEM((CH,), jnp.int32),
        pltpu.VMEM((CH,), jnp.float32),
        pltpu.VMEM((NPT,), jnp.float32),
        pltpu.VMEM_SHARED((N,), jnp.float32),
    ],
)


def _agg_body(row_hbm, col_hbm, xt_hbm, acc_out, row_v, col_v, rows_v, zero_v, acc_sh):
    c = lax.axis_index("c")
    s = lax.axis_index("s")

    def zfill(i, _):
        zero_v[i] = jnp.zeros((16,), jnp.float32)
        return 0

    lax.fori_loop(0, 1024, zfill, 0)

    def zcopy(j, _):
        pltpu.sync_copy(zero_v, acc_sh.at[pl.ds(s * NPT + j * 1024, 1024)])
        return 0

    lax.fori_loop(0, NPT // 1024, zcopy, 0)
    plsc.subcore_barrier()

    ept = E // NS

    def step(k, _):
        base = s * ept + k * CH
        pltpu.sync_copy(row_hbm.at[pl.ds(base, CH)], row_v)
        pltpu.sync_copy(xt_hbm.at[c].at[row_v], rows_v)
        pltpu.sync_copy(col_hbm.at[pl.ds(base, CH)], col_v)
        pltpu.sync_copy(rows_v, acc_sh.at[col_v], add=True)
        return 0

    lax.fori_loop(0, ept // CH, step, 0)
    plsc.subcore_barrier()
    pltpu.sync_copy(acc_sh.at[pl.ds(s * NPT, NPT)], acc_out.at[c].at[pl.ds(s * NPT, NPT)])


_agg = pl.kernel(
    _agg_body,
    out_type=jax.ShapeDtypeStruct((NC, N, H), jnp.float32),
    mesh=_mesh,
    compiler_params=pltpu.CompilerParams(use_tc_tiling_on_sc=False),
    scratch_types=[
        pltpu.VMEM((CH,), jnp.int32),
        pltpu.VMEM((CH,), jnp.int32),
        pltpu.VMEM((CH, H), jnp.float32),
        pltpu.VMEM((1024, H), jnp.float32),
        pltpu.VMEM_SHARED((N, H), jnp.float32),
    ],
)


def _prep(deg2, x):
    BP = 8192

    def body(deg_ref, x_ref, xt_ref, dinv_ref):
        d = deg_ref[0] + deg_ref[1] + 1.0
        dinv = lax.rsqrt(d)[:, None]
        dinv_ref[...] = dinv
        xt_ref[0] = x_ref[:, :H] * dinv
        xt_ref[1] = x_ref[:, H:] * dinv

    return pl.pallas_call(
        body,
        grid=(N // BP,),
        in_specs=[
            pl.BlockSpec((NC, BP), lambda i: (0, i)),
            pl.BlockSpec((BP, D), lambda i: (i, 0)),
        ],
        out_specs=[
            pl.BlockSpec((NC, BP, H), lambda i: (0, i, 0)),
            pl.BlockSpec((BP, 1), lambda i: (i, 0)),
        ],
        out_shape=[
            jax.ShapeDtypeStruct((NC, N, H), jnp.float32),
            jax.ShapeDtypeStruct((N, 1), jnp.float32),
        ],
    )(deg2, x)


def _gcn(acc, xt, dinv, Wg, bg):
    BP = 8192

    def body(acc_ref, xt_ref, dinv_ref, wg_ref, bg_ref, out_ref):
        dinv = dinv_ref[...]
        t0 = (acc_ref[0] + xt_ref[0]) * dinv
        t1 = (acc_ref[1] + xt_ref[1]) * dinv
        h = (
            jnp.dot(t0, wg_ref[0], preferred_element_type=jnp.float32)
            + jnp.dot(t1, wg_ref[1], preferred_element_type=jnp.float32)
            + bg_ref[...]
        )
        out_ref[...] = jnp.maximum(h, 0.0)

    return pl.pallas_call(
        body,
        grid=(N // BP,),
        in_specs=[
            pl.BlockSpec((NC, BP, H), lambda i: (0, i, 0)),
            pl.BlockSpec((NC, BP, H), lambda i: (0, i, 0)),
            pl.BlockSpec((BP, 1), lambda i: (i, 0)),
            pl.BlockSpec((NC, H, 2 * D), lambda i: (0, 0, 0)),
            pl.BlockSpec((1, 2 * D), lambda i: (0, 0)),
        ],
        out_specs=pl.BlockSpec((BP, 2 * D), lambda i: (i, 0)),
        out_shape=jax.ShapeDtypeStruct((N, 2 * D), jnp.float32),
    )(acc, xt, dinv, Wg, bg)


def _mlp(A, W1, b1, W2p, b2p):
    G = N // D
    K = D * 2 * D
    BG = 256

    def body(a_ref, w1_ref, b1_ref, w2_ref, b2_ref, out_ref):
        t = lax.dot_general(
            a_ref[...], w1_ref[...], (((1,), (1,)), ((), ())),
            preferred_element_type=jnp.float32,
        )
        t = jnp.maximum(t + b1_ref[...], 0.0)
        l = (
            lax.dot_general(
                t, w2_ref[...], (((1,), (1,)), ((), ())),
                preferred_element_type=jnp.float32,
            )
            + b2_ref[...]
        )
        m = jnp.max(l, axis=1, keepdims=True)
        e = jnp.exp(l - m)
        out_ref[...] = e / jnp.sum(e, axis=1, keepdims=True)

    return pl.pallas_call(
        body,
        grid=(G // BG,),
        in_specs=[
            pl.BlockSpec((BG, K), lambda i: (i, 0)),
            pl.BlockSpec((2 * K, K), lambda i: (0, 0)),
            pl.BlockSpec((1, 2 * K), lambda i: (0, 0)),
            pl.BlockSpec((16, 2 * K), lambda i: (0, 0)),
            pl.BlockSpec((1, 16), lambda i: (0, 0)),
        ],
        out_specs=pl.BlockSpec((BG, 16), lambda i: (i, 0)),
        out_shape=jax.ShapeDtypeStruct((G, 16), jnp.float32),
    )(A, W1, b1, W2p, b2p)


def kernel(x, edge_index, batch, W_gcn, b_gcn, W1, b1, W2, b2):
    row = edge_index[0]
    col = edge_index[1]
    deg2 = _deg(col)
    xt, dinv = _prep(deg2, x)
    acc = _agg(row, col, xt)
    h = _gcn(acc, xt, dinv, W_gcn.reshape(NC, H, 2 * D), b_gcn.reshape(1, 2 * D))
    A = h.reshape(N // D, D * 2 * D)
    W2p = jnp.pad(W2, ((0, 6), (0, 0)))
    b2p = jnp.concatenate([b2, jnp.full((6,), -1e30, jnp.float32)]).reshape(1, 16)
    out = _mlp(A, W1, b1.reshape(1, -1), W2p, b2p)
    return out[:, :10]
